# Initial kernel scaffold; baseline (speedup 1.0000x reference)
import jax
import jax.numpy as jnp
from jax.experimental import pallas as pl


def kernel(row_idx, col_idx, values):
    raise NotImplementedError("write your pallas kernel here")


# speedup vs baseline: 5.3072x; 5.3072x over previous
import jax
import jax.numpy as jnp
from jax.experimental import pallas as pl
from jax.experimental.pallas import tpu as pltpu

_N_USERS = 4096
_E = 512


def _k1(lo_ref, hi_ref, row_c_ref, col_c_ref, row_w_ref, col_w_ref,
        val_w_ref, pop_ref, niche_ref):
    i = pl.program_id(0)
    lo = lo_ref[i]
    hi = hi_ref[i]
    b0 = lo // _E
    nj = (hi - b0 * _E + _E - 1) // _E
    ri = row_c_ref[...].reshape(_E, 1)
    ci = col_c_ref[...].reshape(_E, 1)
    pi = ci >> 1

    def body(t, acc):
        d, p = acc
        b = b0 + t
        rj = row_w_ref[pl.ds(b, 1)].reshape(1, _E)
        cj = col_w_ref[pl.ds(b, 1)].reshape(1, _E)
        vj = val_w_ref[pl.ds(b, 1)].reshape(1, _E)
        req = ri == rj
        d = d + jnp.sum(jnp.where(req & (ci == cj), vj, 0.0),
                        axis=1, keepdims=True)
        p = p + jnp.sum(jnp.where(req & (pi == (cj >> 1)), vj, 0.0),
                        axis=1, keepdims=True)
        return d, p

    z = jnp.zeros((_E, 1), jnp.float32)
    d, p = jax.lax.fori_loop(0, nj, body, (z, z))
    popv = p * 0.5
    pop_ref[...] = popv.reshape(1, _E, 1)
    niche_ref[...] = (d - popv).reshape(1, _E, 1)


def _k2(lo_ref, hi_ref, row_c_ref, pop_c_ref, nic_c_ref, row_w_ref,
        pop_w_ref, nic_w_ref, up_ref, un_ref):
    i = pl.program_id(0)
    lo = lo_ref[i]
    hi = hi_ref[i]
    b0 = lo // _E
    nj = (hi - b0 * _E + _E - 1) // _E
    ri = row_c_ref[...].reshape(_E, 1)
    xp = pop_c_ref[...].reshape(_E, 1)
    xn = nic_c_ref[...].reshape(_E, 1)
    neg = jnp.float32(-jnp.inf)

    def loop_a(t, acc):
        mp, sp, mn, sn = acc
        b = b0 + t
        rj = row_w_ref[pl.ds(b, 1)].reshape(1, _E)
        pj = pop_w_ref[pl.ds(b, 1)].reshape(1, _E)
        qj = nic_w_ref[pl.ds(b, 1)].reshape(1, _E)
        req = ri == rj
        mp = jnp.maximum(mp, jnp.max(jnp.where(req, pj, neg),
                                     axis=1, keepdims=True))
        sp = sp + jnp.sum(jnp.where(req, pj, 0.0), axis=1, keepdims=True)
        mn = jnp.maximum(mn, jnp.max(jnp.where(req, qj, neg),
                                     axis=1, keepdims=True))
        sn = sn + jnp.sum(jnp.where(req, qj, 0.0), axis=1, keepdims=True)
        return mp, sp, mn, sn

    zf = jnp.zeros((_E, 1), jnp.float32)
    nf = jnp.full((_E, 1), neg, jnp.float32)
    mp, sp, mn, sn = jax.lax.fori_loop(0, nj, loop_a, (nf, zf, nf, zf))

    def loop_b(t, acc):
        zp, zn = acc
        b = b0 + t
        rj = row_w_ref[pl.ds(b, 1)].reshape(1, _E)
        pj = pop_w_ref[pl.ds(b, 1)].reshape(1, _E)
        qj = nic_w_ref[pl.ds(b, 1)].reshape(1, _E)
        req = ri == rj
        zp = zp + jnp.sum(jnp.exp(jnp.where(req, pj - mp, neg)),
                          axis=1, keepdims=True)
        zn = zn + jnp.sum(jnp.exp(jnp.where(req, qj - mn, neg)),
                          axis=1, keepdims=True)
        return zp, zn

    zp, zn = jax.lax.fori_loop(0, nj, loop_b, (zf, zf))
    up = jnp.exp(xp - mp) / zp
    un = jnp.exp(xn - mn) / zn
    up_ref[...] = jnp.where(sp > 0, up, xp).reshape(1, _E, 1)
    un_ref[...] = jnp.where(sn > 0, un, xn).reshape(1, _E, 1)


def kernel(row_idx, col_idx, values):
    nnz = row_idx.shape[0]
    nc = nnz // _E
    row_start = jnp.searchsorted(
        row_idx, jnp.arange(_N_USERS + 1, dtype=jnp.int32)).astype(jnp.int32)
    r_first = row_idx[::_E]
    r_last = row_idx[_E - 1::_E]
    lo = row_start[r_first]
    hi = row_start[r_last + 1]

    row_c = row_idx.reshape(nc, _E, 1)
    col_c = col_idx.reshape(nc, _E, 1)
    row_w = row_idx.reshape(nc, 1, _E)
    col_w = col_idx.reshape(nc, 1, _E)
    val_w = values.reshape(nc, 1, _E)

    smem = pl.BlockSpec(memory_space=pltpu.SMEM)
    chunk = pl.BlockSpec((1, _E, 1), lambda i: (i, 0, 0))
    full = pl.BlockSpec((nc, 1, _E), lambda i: (0, 0, 0))
    out_sh = jax.ShapeDtypeStruct((nc, _E, 1), jnp.float32)

    pop_c, nic_c = pl.pallas_call(
        _k1,
        grid=(nc,),
        in_specs=[smem, smem, chunk, chunk, full, full, full],
        out_specs=[chunk, chunk],
        out_shape=[out_sh, out_sh],
    )(lo, hi, row_c, col_c, row_w, col_w, val_w)

    pop_w = pop_c.reshape(nc, 1, _E)
    nic_w = nic_c.reshape(nc, 1, _E)

    uipg, uing = pl.pallas_call(
        _k2,
        grid=(nc,),
        in_specs=[smem, smem, chunk, chunk, chunk, full, full, full],
        out_specs=[chunk, chunk],
        out_shape=[out_sh, out_sh],
    )(lo, hi, row_c, pop_c, nic_c, row_w, pop_w, nic_w)

    return (uipg.reshape(nnz), uing.reshape(nnz))
